# baseline (speedup 1.0000x reference)
import functools

import jax
import jax.numpy as jnp
from jax import lax
from jax.experimental import pallas as pl
from jax.experimental.pallas import tpu as pltpu
from jax.experimental.pallas import tpu_sc as plsc

NC, NS, L = 2, 16, 16
NW = NC * NS
T, B, N = 8, 128, 4096
R = T * B
RPW = R // NW
GROUP = 8
CHUNKS = N // L
UNROLL = 8


def _dyn_gather(v, idx):
    return lax.gather(
        v,
        idx[:, None],
        dimension_numbers=lax.GatherDimensionNumbers(
            offset_dims=(), collapsed_slice_dims=(0,), start_index_map=(0,)
        ),
        slice_sizes=(1,),
        mode=lax.GatherScatterMode.PROMISE_IN_BOUNDS,
    )


def _tree_sum(acc, iota):
    for st in (8, 4, 2, 1):
        perm = lax.rem(iota + st, L)
        acc = acc + _dyn_gather(acc, perm)
    return acc


def _sc_body(x_hbm, out_hbm, buf_a, buf_b, outv, sem_a, sem_b):
    wid = lax.axis_index("s") * NC + lax.axis_index("c")
    t = wid // (B // RPW)
    r_base = lax.rem(wid, B // RPW) * RPW
    iota = lax.iota(jnp.int32, L)
    one = jnp.full((L,), 1, jnp.int32)
    zero = jnp.full((L,), 0, jnp.int32)

    def copy(g, buf, sem):
        return pltpu.async_copy(
            x_hbm.at[t, pl.ds(r_base + g * GROUP, GROUP), :], buf, sem
        )

    def process(g, buf, carry):
        def row_body(rl, carry):
            c1, c5, c10 = carry
            i = r_base + g * GROUP + rl

            lane = lax.rem(i, L)
            chunk_d = buf[rl, pl.ds(pl.multiple_of(i - lane, L), L)]
            d = _dyn_gather(chunk_d, jnp.full((L,), lane, jnp.int32))

            ilim = jnp.full((L,), i, jnp.int32)
            acc = zero
            for j in range(B // L):
                v = buf[rl, pl.ds(j * L, L)]
                col = iota + (j * L)
                acc = acc + jnp.where((v == d) & (col < ilim), one, zero)

            def chunk(jc, accs):
                a0, a1, a2, a3 = accs
                base = pl.multiple_of(jc * (UNROLL * L), UNROLL * L)
                for q in range(0, UNROLL, 4):
                    v0 = buf[rl, pl.ds(pl.multiple_of(base + q * L, L), L)]
                    v1 = buf[rl, pl.ds(pl.multiple_of(base + (q + 1) * L, L), L)]
                    v2 = buf[rl, pl.ds(pl.multiple_of(base + (q + 2) * L, L), L)]
                    v3 = buf[rl, pl.ds(pl.multiple_of(base + (q + 3) * L, L), L)]
                    a0 = jnp.where(v0 > d, a0 + one, a0)
                    a1 = jnp.where(v1 > d, a1 + one, a1)
                    a2 = jnp.where(v2 > d, a2 + one, a2)
                    a3 = jnp.where(v3 > d, a3 + one, a3)
                return (a0, a1, a2, a3)

            accs = lax.fori_loop(0, CHUNKS // UNROLL, chunk, (acc, zero, zero, zero))
            acc = (accs[0] + accs[1]) + (accs[2] + accs[3])
            rank = _tree_sum(acc, iota)

            c1 = c1 + jnp.where(rank < 1, one, zero)
            c5 = c5 + jnp.where(rank < 5, one, zero)
            c10 = c10 + jnp.where(rank < 10, one, zero)
            return (c1, c5, c10)

        return lax.fori_loop(0, GROUP, row_body, carry)

    carry = (zero, zero, zero)
    h_a = copy(0, buf_a, sem_a)
    h_b = copy(1, buf_b, sem_b)
    h_a.wait()
    carry = process(0, buf_a, carry)
    h_b.wait()
    h_a = copy(2, buf_a, sem_a)
    carry = process(1, buf_b, carry)
    h_a.wait()
    h_b = copy(3, buf_b, sem_b)
    carry = process(2, buf_a, carry)
    h_b.wait()
    carry = process(3, buf_b, carry)
    c1, c5, c10 = carry

    res = (
        jnp.where(iota == 0, c1, zero)
        + jnp.where(iota == 1, c5, zero)
        + jnp.where(iota == 2, c10, zero)
    ).astype(jnp.float32)
    outv[...] = res
    pltpu.sync_copy(outv, out_hbm.at[wid])


@jax.jit
def kernel(input):
    mesh = plsc.VectorSubcoreMesh(
        core_axis_name="c", subcore_axis_name="s", num_cores=NC, num_subcores=NS
    )
    partial = pl.kernel(
        _sc_body,
        out_type=jax.ShapeDtypeStruct((NW, L), jnp.float32),
        mesh=mesh,
        scratch_types=[
            pltpu.VMEM((GROUP, N), jnp.float32),
            pltpu.VMEM((GROUP, N), jnp.float32),
            pltpu.VMEM((L,), jnp.float32),
            pltpu.SemaphoreType.DMA,
            pltpu.SemaphoreType.DMA,
        ],
        compiler_params=pltpu.CompilerParams(use_tc_tiling_on_sc=True),
    )(input)
    sums = jnp.sum(partial[:, :3], axis=0)
    return (sums[0] / R, sums[1] / R, sums[2] / R)


# speedup vs baseline: 1.0014x; 1.0014x over previous
import functools

import jax
import jax.numpy as jnp
from jax import lax
from jax.experimental import pallas as pl
from jax.experimental.pallas import tpu as pltpu
from jax.experimental.pallas import tpu_sc as plsc

NC, NS, L = 2, 16, 16
NW = NC * NS
T, B, N = 8, 128, 4096
R = T * B
RPW = R // NW
GROUP = 8
CHUNKS = N // L
UNROLL = 8


def _dyn_gather(v, idx):
    return lax.gather(
        v,
        idx[:, None],
        dimension_numbers=lax.GatherDimensionNumbers(
            offset_dims=(), collapsed_slice_dims=(0,), start_index_map=(0,)
        ),
        slice_sizes=(1,),
        mode=lax.GatherScatterMode.PROMISE_IN_BOUNDS,
    )


def _tree_sum(acc, iota):
    for st in (8, 4, 2, 1):
        perm = lax.rem(iota + st, L)
        acc = acc + _dyn_gather(acc, perm)
    return acc


def _sc_body(x_hbm, out_hbm, buf_a, buf_b, outv, sem_a, sem_b):
    wid = lax.axis_index("s") * NC + lax.axis_index("c")
    t = wid // (B // RPW)
    r_base = lax.rem(wid, B // RPW) * RPW
    iota = lax.iota(jnp.int32, L)
    one = jnp.full((L,), 1, jnp.int32)
    zero = jnp.full((L,), 0, jnp.int32)

    def copy(g, buf, sem):
        return pltpu.async_copy(
            x_hbm.at[t, pl.ds(r_base + g * GROUP, GROUP), :], buf, sem
        )

    def process(g, buf, carry):
        def row_body(rl, carry):
            c1, c5, c10 = carry
            i = r_base + g * GROUP + rl

            lane = lax.rem(i, L)
            chunk_d = buf[rl, pl.ds(pl.multiple_of(i - lane, L), L)]
            d = _dyn_gather(chunk_d, jnp.full((L,), lane, jnp.int32))

            ilim = jnp.full((L,), i, jnp.int32)
            acc = zero
            for j in range(B // L):
                v = buf[rl, pl.ds(j * L, L)]
                col = iota + (j * L)
                acc = acc + jnp.where((v == d) & (col < ilim), one, zero)

            def chunk(c0, accs):
                a0, a1, a2, a3 = accs
                base = pl.multiple_of(c0, UNROLL * L)
                for q in range(0, UNROLL, 4):
                    v0 = buf[rl, pl.ds(pl.multiple_of(base + q * L, L), L)]
                    v1 = buf[rl, pl.ds(pl.multiple_of(base + (q + 1) * L, L), L)]
                    v2 = buf[rl, pl.ds(pl.multiple_of(base + (q + 2) * L, L), L)]
                    v3 = buf[rl, pl.ds(pl.multiple_of(base + (q + 3) * L, L), L)]
                    a0 = jnp.where(v0 > d, a0 + one, a0)
                    a1 = jnp.where(v1 > d, a1 + one, a1)
                    a2 = jnp.where(v2 > d, a2 + one, a2)
                    a3 = jnp.where(v3 > d, a3 + one, a3)
                return (a0, a1, a2, a3)

            accs = plsc.parallel_loop(
                0, N, step=UNROLL * L, unroll=2, carry=(acc, zero, zero, zero)
            )(chunk)
            acc = (accs[0] + accs[1]) + (accs[2] + accs[3])
            rank = _tree_sum(acc, iota)

            c1 = c1 + jnp.where(rank < 1, one, zero)
            c5 = c5 + jnp.where(rank < 5, one, zero)
            c10 = c10 + jnp.where(rank < 10, one, zero)
            return (c1, c5, c10)

        return lax.fori_loop(0, GROUP, row_body, carry)

    carry = (zero, zero, zero)
    h_a = copy(0, buf_a, sem_a)
    h_b = copy(1, buf_b, sem_b)
    h_a.wait()
    carry = process(0, buf_a, carry)
    h_b.wait()
    h_a = copy(2, buf_a, sem_a)
    carry = process(1, buf_b, carry)
    h_a.wait()
    h_b = copy(3, buf_b, sem_b)
    carry = process(2, buf_a, carry)
    h_b.wait()
    carry = process(3, buf_b, carry)
    c1, c5, c10 = carry

    res = (
        jnp.where(iota == 0, c1, zero)
        + jnp.where(iota == 1, c5, zero)
        + jnp.where(iota == 2, c10, zero)
    ).astype(jnp.float32)
    outv[...] = res
    pltpu.sync_copy(outv, out_hbm.at[wid])


@jax.jit
def kernel(input):
    mesh = plsc.VectorSubcoreMesh(
        core_axis_name="c", subcore_axis_name="s", num_cores=NC, num_subcores=NS
    )
    partial = pl.kernel(
        _sc_body,
        out_type=jax.ShapeDtypeStruct((NW, L), jnp.float32),
        mesh=mesh,
        scratch_types=[
            pltpu.VMEM((GROUP, N), jnp.float32),
            pltpu.VMEM((GROUP, N), jnp.float32),
            pltpu.VMEM((L,), jnp.float32),
            pltpu.SemaphoreType.DMA,
            pltpu.SemaphoreType.DMA,
        ],
        compiler_params=pltpu.CompilerParams(use_tc_tiling_on_sc=True),
    )(input)
    sums = jnp.sum(partial[:, :3], axis=0)
    return (sums[0] / R, sums[1] / R, sums[2] / R)


# speedup vs baseline: 1.0998x; 1.0983x over previous
import jax
import jax.numpy as jnp
from jax import lax
from jax.experimental import pallas as pl
from jax.experimental.pallas import tpu as pltpu
from jax.experimental.pallas import tpu_sc as plsc

NC, NS, L = 2, 16, 16
NW = NC * NS
T, B, N = 8, 128, 4096
R = T * B
T_TC = 4
T_SC = T - T_TC
RPW = T_SC * B // NW
GROUP = 8
CHUNKS = N // L
UNROLL = 8
WPT = B // RPW


def _dyn_gather(v, idx):
    return lax.gather(
        v,
        idx[:, None],
        dimension_numbers=lax.GatherDimensionNumbers(
            offset_dims=(), collapsed_slice_dims=(0,), start_index_map=(0,)
        ),
        slice_sizes=(1,),
        mode=lax.GatherScatterMode.PROMISE_IN_BOUNDS,
    )


def _tree_sum(acc, iota):
    for st in (8, 4, 2, 1):
        perm = lax.rem(iota + st, L)
        acc = acc + _dyn_gather(acc, perm)
    return acc


def _sc_body(x_hbm, out_hbm, buf_a, buf_b, outv, sem_a, sem_b):
    wid = lax.axis_index("s") * NC + lax.axis_index("c")
    t = T_TC + wid // WPT
    r_base = lax.rem(wid, WPT) * RPW
    iota = lax.iota(jnp.int32, L)
    one = jnp.full((L,), 1, jnp.int32)
    zero = jnp.full((L,), 0, jnp.int32)

    def copy(g, buf, sem):
        return pltpu.async_copy(
            x_hbm.at[t, pl.ds(r_base + g * GROUP, GROUP), :], buf, sem
        )

    def process(g, buf, carry):
        def row_body(rl, carry):
            c1, c5, c10 = carry
            i = r_base + g * GROUP + rl

            lane = lax.rem(i, L)
            chunk_d = buf[rl, pl.ds(pl.multiple_of(i - lane, L), L)]
            d = _dyn_gather(chunk_d, jnp.full((L,), lane, jnp.int32))

            ilim = jnp.full((L,), i, jnp.int32)
            acc = zero
            for j in range(B // L):
                v = buf[rl, pl.ds(j * L, L)]
                col = iota + (j * L)
                acc = acc + jnp.where((v == d) & (col < ilim), one, zero)

            def chunk(c0, accs):
                a0, a1, a2, a3 = accs
                base = pl.multiple_of(c0, UNROLL * L)
                for q in range(0, UNROLL, 4):
                    v0 = buf[rl, pl.ds(pl.multiple_of(base + q * L, L), L)]
                    v1 = buf[rl, pl.ds(pl.multiple_of(base + (q + 1) * L, L), L)]
                    v2 = buf[rl, pl.ds(pl.multiple_of(base + (q + 2) * L, L), L)]
                    v3 = buf[rl, pl.ds(pl.multiple_of(base + (q + 3) * L, L), L)]
                    a0 = jnp.where(v0 > d, a0 + one, a0)
                    a1 = jnp.where(v1 > d, a1 + one, a1)
                    a2 = jnp.where(v2 > d, a2 + one, a2)
                    a3 = jnp.where(v3 > d, a3 + one, a3)
                return (a0, a1, a2, a3)

            accs = plsc.parallel_loop(
                0, N, step=UNROLL * L, unroll=2, carry=(acc, zero, zero, zero)
            )(chunk)
            acc = (accs[0] + accs[1]) + (accs[2] + accs[3])
            rank = _tree_sum(acc, iota)

            c1 = c1 + jnp.where(rank < 1, one, zero)
            c5 = c5 + jnp.where(rank < 5, one, zero)
            c10 = c10 + jnp.where(rank < 10, one, zero)
            return (c1, c5, c10)

        return lax.fori_loop(0, GROUP, row_body, carry)

    carry = (zero, zero, zero)
    h_a = copy(0, buf_a, sem_a)
    h_b = copy(1, buf_b, sem_b)
    h_a.wait()
    carry = process(0, buf_a, carry)
    h_b.wait()
    carry = process(1, buf_b, carry)
    c1, c5, c10 = carry

    res = (
        jnp.where(iota == 0, c1, zero)
        + jnp.where(iota == 1, c5, zero)
        + jnp.where(iota == 2, c10, zero)
    ).astype(jnp.float32)
    outv[...] = res
    pltpu.sync_copy(outv, out_hbm.at[wid])


def _tc_body(x_ref, out_ref):
    x = x_ref[0]
    row = lax.broadcasted_iota(jnp.int32, (B, N), 0)
    col = lax.broadcasted_iota(jnp.int32, (B, N), 1)
    diag = col == row
    d = jnp.sum(jnp.where(diag, x, jnp.float32(0.0)), axis=1, keepdims=True)
    gt = jnp.sum((x > d).astype(jnp.int32), axis=1, keepdims=True)
    eq = jnp.sum(((x == d) & (col < row)).astype(jnp.int32), axis=1, keepdims=True)
    rank = gt + eq
    h1 = jnp.sum((rank < 1).astype(jnp.float32))
    h5 = jnp.sum((rank < 5).astype(jnp.float32))
    h10 = jnp.sum((rank < 10).astype(jnp.float32))
    lanes = lax.broadcasted_iota(jnp.int32, (1, 1, B), 2)
    out_ref[...] = jnp.where(
        lanes == 0, h1, jnp.where(lanes == 1, h5, jnp.where(lanes == 2, h10, 0.0))
    )


@jax.jit
def kernel(input):
    mesh = plsc.VectorSubcoreMesh(
        core_axis_name="c", subcore_axis_name="s", num_cores=NC, num_subcores=NS
    )
    sc_partial = pl.kernel(
        _sc_body,
        out_type=jax.ShapeDtypeStruct((NW, L), jnp.float32),
        mesh=mesh,
        scratch_types=[
            pltpu.VMEM((GROUP, N), jnp.float32),
            pltpu.VMEM((GROUP, N), jnp.float32),
            pltpu.VMEM((L,), jnp.float32),
            pltpu.SemaphoreType.DMA,
            pltpu.SemaphoreType.DMA,
        ],
        compiler_params=pltpu.CompilerParams(use_tc_tiling_on_sc=True),
    )(input)

    tc_partial = pl.pallas_call(
        _tc_body,
        grid=(T_TC,),
        in_specs=[pl.BlockSpec((1, B, N), lambda t: (t, 0, 0))],
        out_specs=pl.BlockSpec((1, 1, B), lambda t: (t, 0, 0)),
        out_shape=jax.ShapeDtypeStruct((T_TC, 1, B), jnp.float32),
    )(input)

    sums = jnp.sum(sc_partial[:, :3], axis=0) + jnp.sum(tc_partial[:, 0, :3], axis=0)
    return (sums[0] / R, sums[1] / R, sums[2] / R)


# speedup vs baseline: 1.1540x; 1.0493x over previous
import jax
import jax.numpy as jnp
from jax import lax
from jax.experimental import pallas as pl
from jax.experimental.pallas import tpu as pltpu
from jax.experimental.pallas import tpu_sc as plsc

NC, NS, L = 2, 16, 16
NW = NC * NS
T, B, N = 8, 128, 4096
R = T * B
T_TC = 6
T_SC = T - T_TC
RPW = T_SC * B // NW
GROUP = 8
CHUNKS = N // L
UNROLL = 8
WPT = B // RPW


def _dyn_gather(v, idx):
    return lax.gather(
        v,
        idx[:, None],
        dimension_numbers=lax.GatherDimensionNumbers(
            offset_dims=(), collapsed_slice_dims=(0,), start_index_map=(0,)
        ),
        slice_sizes=(1,),
        mode=lax.GatherScatterMode.PROMISE_IN_BOUNDS,
    )


def _tree_sum(acc, iota):
    for st in (8, 4, 2, 1):
        perm = lax.rem(iota + st, L)
        acc = acc + _dyn_gather(acc, perm)
    return acc


def _sc_body(x_hbm, out_hbm, buf_a, buf_b, outv, sem_a, sem_b):
    wid = lax.axis_index("s") * NC + lax.axis_index("c")
    t = T_TC + wid // WPT
    r_base = lax.rem(wid, WPT) * RPW
    iota = lax.iota(jnp.int32, L)
    one = jnp.full((L,), 1, jnp.int32)
    zero = jnp.full((L,), 0, jnp.int32)

    def copy(g, buf, sem):
        return pltpu.async_copy(
            x_hbm.at[t, pl.ds(r_base + g * GROUP, GROUP), :], buf, sem
        )

    def process(g, buf, carry):
        def row_body(rl, carry):
            c1, c5, c10 = carry
            i = r_base + g * GROUP + rl

            lane = lax.rem(i, L)
            chunk_d = buf[rl, pl.ds(pl.multiple_of(i - lane, L), L)]
            d = _dyn_gather(chunk_d, jnp.full((L,), lane, jnp.int32))

            ilim = jnp.full((L,), i, jnp.int32)
            acc = zero
            for j in range(B // L):
                v = buf[rl, pl.ds(j * L, L)]
                col = iota + (j * L)
                acc = acc + jnp.where((v == d) & (col < ilim), one, zero)

            def chunk(c0, accs):
                a0, a1, a2, a3 = accs
                base = pl.multiple_of(c0, UNROLL * L)
                for q in range(0, UNROLL, 4):
                    v0 = buf[rl, pl.ds(pl.multiple_of(base + q * L, L), L)]
                    v1 = buf[rl, pl.ds(pl.multiple_of(base + (q + 1) * L, L), L)]
                    v2 = buf[rl, pl.ds(pl.multiple_of(base + (q + 2) * L, L), L)]
                    v3 = buf[rl, pl.ds(pl.multiple_of(base + (q + 3) * L, L), L)]
                    a0 = jnp.where(v0 > d, a0 + one, a0)
                    a1 = jnp.where(v1 > d, a1 + one, a1)
                    a2 = jnp.where(v2 > d, a2 + one, a2)
                    a3 = jnp.where(v3 > d, a3 + one, a3)
                return (a0, a1, a2, a3)

            accs = plsc.parallel_loop(
                0, N, step=UNROLL * L, unroll=2, carry=(acc, zero, zero, zero)
            )(chunk)
            acc = (accs[0] + accs[1]) + (accs[2] + accs[3])
            rank = _tree_sum(acc, iota)

            c1 = c1 + jnp.where(rank < 1, one, zero)
            c5 = c5 + jnp.where(rank < 5, one, zero)
            c10 = c10 + jnp.where(rank < 10, one, zero)
            return (c1, c5, c10)

        return lax.fori_loop(0, GROUP, row_body, carry)

    carry = (zero, zero, zero)
    if RPW // GROUP >= 2:
        h_a = copy(0, buf_a, sem_a)
        h_b = copy(1, buf_b, sem_b)
        h_a.wait()
        carry = process(0, buf_a, carry)
        h_b.wait()
        carry = process(1, buf_b, carry)
    else:
        h_a = copy(0, buf_a, sem_a)
        h_a.wait()
        carry = process(0, buf_a, carry)
    c1, c5, c10 = carry

    res = (
        jnp.where(iota == 0, c1, zero)
        + jnp.where(iota == 1, c5, zero)
        + jnp.where(iota == 2, c10, zero)
    ).astype(jnp.float32)
    outv[...] = res
    pltpu.sync_copy(outv, out_hbm.at[wid])


def _tc_body(x_ref, out_ref):
    x = x_ref[0]
    row = lax.broadcasted_iota(jnp.int32, (B, N), 0)
    col = lax.broadcasted_iota(jnp.int32, (B, N), 1)
    diag = col == row
    d = jnp.sum(jnp.where(diag, x, jnp.float32(0.0)), axis=1, keepdims=True)
    gt = jnp.sum((x > d).astype(jnp.int32), axis=1, keepdims=True)
    eq = jnp.sum(((x == d) & (col < row)).astype(jnp.int32), axis=1, keepdims=True)
    rank = gt + eq
    h1 = jnp.sum((rank < 1).astype(jnp.float32))
    h5 = jnp.sum((rank < 5).astype(jnp.float32))
    h10 = jnp.sum((rank < 10).astype(jnp.float32))
    lanes = lax.broadcasted_iota(jnp.int32, (1, 1, B), 2)
    out_ref[...] = jnp.where(
        lanes == 0, h1, jnp.where(lanes == 1, h5, jnp.where(lanes == 2, h10, 0.0))
    )


@jax.jit
def kernel(input):
    mesh = plsc.VectorSubcoreMesh(
        core_axis_name="c", subcore_axis_name="s", num_cores=NC, num_subcores=NS
    )
    sc_partial = pl.kernel(
        _sc_body,
        out_type=jax.ShapeDtypeStruct((NW, L), jnp.float32),
        mesh=mesh,
        scratch_types=[
            pltpu.VMEM((GROUP, N), jnp.float32),
            pltpu.VMEM((GROUP, N), jnp.float32),
            pltpu.VMEM((L,), jnp.float32),
            pltpu.SemaphoreType.DMA,
            pltpu.SemaphoreType.DMA,
        ],
        compiler_params=pltpu.CompilerParams(use_tc_tiling_on_sc=True),
    )(input)

    tc_partial = pl.pallas_call(
        _tc_body,
        grid=(T_TC,),
        in_specs=[pl.BlockSpec((1, B, N), lambda t: (t, 0, 0))],
        out_specs=pl.BlockSpec((1, 1, B), lambda t: (t, 0, 0)),
        out_shape=jax.ShapeDtypeStruct((T_TC, 1, B), jnp.float32),
    )(input)

    sums = jnp.sum(sc_partial[:, :3], axis=0) + jnp.sum(tc_partial[:, 0, :3], axis=0)
    return (sums[0] / R, sums[1] / R, sums[2] / R)
